# baseline (speedup 1.0000x reference)
import functools

import jax
import jax.numpy as jnp
from jax import lax
from jax.experimental import pallas as pl
from jax.experimental.pallas import tpu as pltpu
from jax.experimental.pallas import tpu_sc as plsc

BATCH = 16384
VOCAB = 1000000
EMBED = 32
HID = 10
HPAD = 16
PGRP = 8
CMOD = 131072
SBLK = 8192
SGRID = CMOD // SBLK
LASTB = (VOCAB - 1) // SBLK

NC = 2
NS = 16
NW = NC * NS
BPW = BATCH // NW
CHUNK = 128
NCHUNK = BPW // CHUNK


def _scan_body(*refs):
    ins = refs[:2 * PGRP]
    wc, bc, wp, bp_, ok_ref, oc_ref, op_ref = refs[2 * PGRP:]
    f32 = jnp.float32
    ok = ok_ref[...]

    def piece(r, j):
        x = r[...]
        return x * ok if j == PGRP - 1 else x

    xc = jnp.concatenate([piece(ins[j], j) for j in range(PGRP)], axis=0)
    xp = jnp.concatenate([piece(ins[PGRP + j], j) for j in range(PGRP)],
                         axis=0)
    dn = (((0,), (0,)), ((), ()))
    bf = jnp.bfloat16
    zc = lax.dot_general(xc.astype(bf), wc[...].astype(bf), dn,
                         preferred_element_type=f32)
    zp = lax.dot_general(xp.astype(bf), wp[...].astype(bf), dn,
                         preferred_element_type=f32)
    oc_ref[...] = jnp.maximum(zc + bc[...], 0.0)
    op_ref[...] = jnp.maximum(zp + bp_[...], 0.0)


def _scan(custT, prodT, Wbig_c, bbig_c, Wbig_p, bbig_p, okmask):
    in_specs = []
    for t in range(2):
        for j in range(PGRP):
            in_specs.append(pl.BlockSpec(
                (EMBED, SBLK),
                lambda i, j=j: (0, jnp.minimum(SGRID * j + i, LASTB))))
    full = lambda a: pl.BlockSpec(a.shape, lambda i: (0, 0))
    in_specs += [full(Wbig_c), full(bbig_c), full(Wbig_p), full(bbig_p),
                 pl.BlockSpec((1, SBLK), lambda i: (i, 0))]
    out_spec = pl.BlockSpec((SBLK, PGRP * HPAD), lambda i: (i, 0))
    mesh = pltpu.create_tensorcore_mesh("x")

    @functools.partial(
        pl.kernel,
        mesh=mesh,
        out_type=[
            jax.ShapeDtypeStruct((CMOD, PGRP * HPAD), jnp.float32),
            jax.ShapeDtypeStruct((CMOD, PGRP * HPAD), jnp.float32),
        ],
    )
    def k(ct_hbm, pt_hbm, wc_hbm, bc_hbm, wp_hbm, bp_hbm, ok_hbm,
          ozc_hbm, ozp_hbm):
        pltpu.emit_pipeline(
            _scan_body,
            grid=(SGRID,),
            in_specs=in_specs,
            out_specs=[out_spec, out_spec],
            core_axis_name="x",
            dimension_semantics=(pltpu.PARALLEL,),
        )(*([ct_hbm] * PGRP + [pt_hbm] * PGRP
            + [wc_hbm, bc_hbm, wp_hbm, bp_hbm, ok_hbm]),
          ozc_hbm, ozp_hbm)

    return k(custT, prodT, Wbig_c, bbig_c, Wbig_p, bbig_p, okmask)


def _sc_gather(zc, zp, ip, ic):
    mesh = plsc.VectorSubcoreMesh(core_axis_name="c", subcore_axis_name="s")
    BLK = PGRP * HPAD

    @functools.partial(
        pl.kernel,
        mesh=mesh,
        out_type=[
            jax.ShapeDtypeStruct((BATCH, BLK), jnp.float32),
            jax.ShapeDtypeStruct((BATCH, BLK), jnp.float32),
        ],
        scratch_types=[
            pltpu.VMEM((NCHUNK, CHUNK), jnp.int32),
            pltpu.VMEM((NCHUNK, CHUNK), jnp.int32),
            pltpu.VMEM((CHUNK, BLK), jnp.float32),
            pltpu.VMEM((CHUNK, BLK), jnp.float32),
            pltpu.VMEM((CHUNK, BLK), jnp.float32),
            pltpu.VMEM((CHUNK, BLK), jnp.float32),
            pltpu.SemaphoreType.DMA,
            pltpu.SemaphoreType.DMA,
            pltpu.SemaphoreType.DMA,
            pltpu.SemaphoreType.DMA,
        ],
    )
    def k(zc_hbm, zp_hbm, ip_hbm, ic_hbm, oc_hbm, op_hbm,
          ipv, icv, pv0, pv1, cv0, cv1, sp0, sp1, sc0, sc1):
        wid = lax.axis_index("s") * NC + lax.axis_index("c")
        base = wid * BPW
        pltpu.sync_copy(ip_hbm.at[wid], ipv)
        pltpu.sync_copy(ic_hbm.at[wid], icv)
        pbuf, cbuf = (pv0, pv1), (cv0, cv1)
        psem, csem = (sp0, sp1), (sc0, sc1)

        def start(j):
            s = j & 1
            return (
                pltpu.async_copy(zc_hbm.at[ipv.at[j]], pbuf[s], psem[s]),
                pltpu.async_copy(zp_hbm.at[icv.at[j]], cbuf[s], csem[s]),
            )

        cps = [start(0)]
        for j in range(NCHUNK):
            if j + 1 < NCHUNK:
                cps.append(start(j + 1))
            cps[j][0].wait()
            cps[j][1].wait()
            s = j & 1
            dst = pl.ds(base + j * CHUNK, CHUNK)
            pltpu.sync_copy(pbuf[s], oc_hbm.at[dst])
            pltpu.sync_copy(cbuf[s], op_hbm.at[dst])

    return k(zc, zp, ip, ic)


_MLP_BS = 2048


def _mlp_body(gp_ref, gc_ref, pp_ref, pc_ref, w2a, w2b, b2, wo, bo, o_ref):
    f32 = jnp.float32
    lanegrp = jax.lax.broadcasted_iota(jnp.int32, (1, PGRP * HPAD), 1) // HPAD
    gpm = gp_ref[...].astype(f32) * (lanegrp == pp_ref[...]).astype(f32)
    gcm = gc_ref[...].astype(f32) * (lanegrp == pc_ref[...]).astype(f32)
    h2 = jnp.maximum(
        jnp.dot(gpm, w2a[...], preferred_element_type=f32)
        + jnp.dot(gcm, w2b[...], preferred_element_type=f32) + b2[...], 0.0)
    z = jnp.dot(h2, wo[...], preferred_element_type=f32) + bo[...]
    o_ref[...] = jax.nn.sigmoid(z)


def _mlp(gp, gc, pp, pc, W2a, W2b, b2, Wo, bo):
    grid = (BATCH // _MLP_BS,)
    full = lambda a: pl.BlockSpec(a.shape, lambda i: (0, 0))
    return pl.pallas_call(
        _mlp_body,
        grid=grid,
        in_specs=[
            pl.BlockSpec((_MLP_BS, PGRP * HPAD), lambda i: (i, 0)),
            pl.BlockSpec((_MLP_BS, PGRP * HPAD), lambda i: (i, 0)),
            pl.BlockSpec((_MLP_BS, 1), lambda i: (i, 0)),
            pl.BlockSpec((_MLP_BS, 1), lambda i: (i, 0)),
            full(W2a), full(W2b), full(b2), full(Wo), full(bo),
        ],
        out_specs=pl.BlockSpec((_MLP_BS, 1), lambda i: (i, 0)),
        out_shape=jax.ShapeDtypeStruct((BATCH, 1), jnp.float32),
    )(gp, gc, pp, pc, W2a, W2b, b2, Wo, bo)


def _bigw(W, b):
    W16 = jnp.pad(W, ((0, 0), (0, HPAD - HID)))
    b16 = jnp.pad(b, (0, HPAD - HID))
    Wbig = jnp.kron(jnp.eye(PGRP, dtype=jnp.float32), W16)
    bbig = jnp.tile(b16, PGRP).reshape(1, PGRP * HPAD)
    return Wbig, bbig


def kernel(X, encoded_customers, encoded_products, W_prod, b_prod,
           W_cust, b_cust, W_fc2, b_fc2, W_out, b_out):
    custT = encoded_customers.T
    prodT = encoded_products.T
    Wbig_c, bbig_c = _bigw(W_prod, b_prod)
    Wbig_p, bbig_p = _bigw(W_cust, b_cust)
    lastblk = jnp.minimum(
        SGRID * (PGRP - 1) + jnp.arange(SGRID, dtype=jnp.int32), LASTB)
    okmask = ((lastblk[:, None] * SBLK
               + jnp.arange(SBLK, dtype=jnp.int32)[None, :]) < VOCAB
              ).astype(jnp.float32)
    zc, zp = _scan(custT, prodT, Wbig_c, bbig_c, Wbig_p, bbig_p, okmask)

    rp = X[:, 0].astype(jnp.int32)
    rc = X[:, 1].astype(jnp.int32)
    ip = (rp & (CMOD - 1)).reshape(NW, NCHUNK, CHUNK)
    ic = (rc & (CMOD - 1)).reshape(NW, NCHUNK, CHUNK)
    pp = (rp >> 17).reshape(BATCH, 1)
    pc = (rc >> 17).reshape(BATCH, 1)

    gp, gc = _sc_gather(zc, zp, ip, ic)
    rep = lambda W: jnp.tile(jnp.pad(W, ((0, HPAD - HID), (0, 0))), (PGRP, 1))
    out = _mlp(
        gp, gc, pp, pc,
        rep(W_fc2[:HID]), rep(W_fc2[HID:]), b_fc2.reshape(1, HID),
        W_out, b_out.reshape(1, 1),
    )
    return out


# speedup vs baseline: 1.0216x; 1.0216x over previous
import functools

import jax
import jax.numpy as jnp
from jax import lax
from jax.experimental import pallas as pl
from jax.experimental.pallas import tpu as pltpu
from jax.experimental.pallas import tpu_sc as plsc

BATCH = 16384
VOCAB = 1000000
EMBED = 32
HID = 10
HPAD = 16
PGRP = 8
CMOD = 131072
SBLK = 8192
SGRID = CMOD // SBLK
LASTB = (VOCAB - 1) // SBLK

NC = 2
NS = 16
NW = NC * NS
BPW = BATCH // NW
CHUNK = 128
NCHUNK = BPW // CHUNK


def _scan_body(*refs):
    ins = refs[:2 * PGRP]
    wc, bc, wp, bp_, oc_ref, op_ref = refs[2 * PGRP:]
    f32 = jnp.float32
    blk = jnp.minimum(SGRID * (PGRP - 1) + pl.program_id(0), LASTB)
    lane = jax.lax.broadcasted_iota(jnp.int32, (1, SBLK), 1)
    ok = (blk * SBLK + lane) < VOCAB

    def piece(r, j):
        x = r[...]
        return jnp.where(ok, x, 0.0) if j == PGRP - 1 else x

    xc = jnp.concatenate([piece(ins[j], j) for j in range(PGRP)], axis=0)
    xp = jnp.concatenate([piece(ins[PGRP + j], j) for j in range(PGRP)],
                         axis=0)
    dn = (((0,), (0,)), ((), ()))
    bf = jnp.bfloat16
    zc = lax.dot_general(xc.astype(bf), wc[...].astype(bf), dn,
                         preferred_element_type=f32)
    zp = lax.dot_general(xp.astype(bf), wp[...].astype(bf), dn,
                         preferred_element_type=f32)
    oc_ref[...] = jnp.maximum(zc + bc[...], 0.0)
    op_ref[...] = jnp.maximum(zp + bp_[...], 0.0)


def _scan(custT, prodT, Wbig_c, bbig_c, Wbig_p, bbig_p):
    in_specs = []
    for t in range(2):
        for j in range(PGRP):
            in_specs.append(pl.BlockSpec(
                (EMBED, SBLK),
                lambda i, j=j: (0, jnp.minimum(SGRID * j + i, LASTB))))
    full = lambda a: pl.BlockSpec(a.shape, lambda i: (0, 0))
    in_specs += [full(Wbig_c), full(bbig_c), full(Wbig_p), full(bbig_p)]
    out_spec = pl.BlockSpec((SBLK, PGRP * HPAD), lambda i: (i, 0))
    return pl.pallas_call(
        _scan_body,
        grid=(SGRID,),
        in_specs=in_specs,
        out_specs=[out_spec, out_spec],
        out_shape=[
            jax.ShapeDtypeStruct((CMOD, PGRP * HPAD), jnp.float32),
            jax.ShapeDtypeStruct((CMOD, PGRP * HPAD), jnp.float32),
        ],
        compiler_params=pltpu.CompilerParams(
            dimension_semantics=("parallel",),
            vmem_limit_bytes=60 * 1024 * 1024),
    )(*([custT] * PGRP + [prodT] * PGRP + [Wbig_c, bbig_c, Wbig_p, bbig_p]))


def _sc_gather(zc, zp, ip, ic):
    mesh = plsc.VectorSubcoreMesh(core_axis_name="c", subcore_axis_name="s")
    BLK = PGRP * HPAD

    @functools.partial(
        pl.kernel,
        mesh=mesh,
        out_type=[
            jax.ShapeDtypeStruct((BATCH, BLK), jnp.float32),
            jax.ShapeDtypeStruct((BATCH, BLK), jnp.float32),
        ],
        scratch_types=[
            pltpu.VMEM((NCHUNK, CHUNK), jnp.int32),
            pltpu.VMEM((NCHUNK, CHUNK), jnp.int32),
            pltpu.VMEM((CHUNK, BLK), jnp.float32),
            pltpu.VMEM((CHUNK, BLK), jnp.float32),
            pltpu.VMEM((CHUNK, BLK), jnp.float32),
            pltpu.VMEM((CHUNK, BLK), jnp.float32),
            pltpu.SemaphoreType.DMA,
            pltpu.SemaphoreType.DMA,
            pltpu.SemaphoreType.DMA,
            pltpu.SemaphoreType.DMA,
        ],
    )
    def k(zc_hbm, zp_hbm, ip_hbm, ic_hbm, oc_hbm, op_hbm,
          ipv, icv, pv0, pv1, cv0, cv1, sp0, sp1, sc0, sc1):
        wid = lax.axis_index("s") * NC + lax.axis_index("c")
        base = wid * BPW
        pltpu.sync_copy(ip_hbm.at[wid], ipv)
        pltpu.sync_copy(ic_hbm.at[wid], icv)
        pbuf, cbuf = (pv0, pv1), (cv0, cv1)
        psem, csem = (sp0, sp1), (sc0, sc1)

        def start(j):
            s = j & 1
            return (
                pltpu.async_copy(zc_hbm.at[ipv.at[j]], pbuf[s], psem[s]),
                pltpu.async_copy(zp_hbm.at[icv.at[j]], cbuf[s], csem[s]),
            )

        cps = [start(0)]
        for j in range(NCHUNK):
            if j + 1 < NCHUNK:
                cps.append(start(j + 1))
            cps[j][0].wait()
            cps[j][1].wait()
            s = j & 1
            dst = pl.ds(base + j * CHUNK, CHUNK)
            pltpu.sync_copy(pbuf[s], oc_hbm.at[dst])
            pltpu.sync_copy(cbuf[s], op_hbm.at[dst])

    return k(zc, zp, ip, ic)


_MLP_BS = 2048


def _mlp_body(gp_ref, gc_ref, pp_ref, pc_ref, w2a, w2b, b2, wo, bo, o_ref):
    f32 = jnp.float32
    lanegrp = jax.lax.broadcasted_iota(jnp.int32, (1, PGRP * HPAD), 1) // HPAD
    gpm = gp_ref[...].astype(f32) * (lanegrp == pp_ref[...]).astype(f32)
    gcm = gc_ref[...].astype(f32) * (lanegrp == pc_ref[...]).astype(f32)
    h2 = jnp.maximum(
        jnp.dot(gpm, w2a[...], preferred_element_type=f32)
        + jnp.dot(gcm, w2b[...], preferred_element_type=f32) + b2[...], 0.0)
    z = jnp.dot(h2, wo[...], preferred_element_type=f32) + bo[...]
    o_ref[...] = jax.nn.sigmoid(z)


def _mlp(gp, gc, pp, pc, W2a, W2b, b2, Wo, bo):
    grid = (BATCH // _MLP_BS,)
    full = lambda a: pl.BlockSpec(a.shape, lambda i: (0, 0))
    return pl.pallas_call(
        _mlp_body,
        grid=grid,
        in_specs=[
            pl.BlockSpec((_MLP_BS, PGRP * HPAD), lambda i: (i, 0)),
            pl.BlockSpec((_MLP_BS, PGRP * HPAD), lambda i: (i, 0)),
            pl.BlockSpec((_MLP_BS, 1), lambda i: (i, 0)),
            pl.BlockSpec((_MLP_BS, 1), lambda i: (i, 0)),
            full(W2a), full(W2b), full(b2), full(Wo), full(bo),
        ],
        out_specs=pl.BlockSpec((_MLP_BS, 1), lambda i: (i, 0)),
        out_shape=jax.ShapeDtypeStruct((BATCH, 1), jnp.float32),
    )(gp, gc, pp, pc, W2a, W2b, b2, Wo, bo)


def _bigw(W, b):
    W16 = jnp.pad(W, ((0, 0), (0, HPAD - HID)))
    b16 = jnp.pad(b, (0, HPAD - HID))
    Wbig = jnp.kron(jnp.eye(PGRP, dtype=jnp.float32), W16)
    bbig = jnp.tile(b16, PGRP).reshape(1, PGRP * HPAD)
    return Wbig, bbig


def kernel(X, encoded_customers, encoded_products, W_prod, b_prod,
           W_cust, b_cust, W_fc2, b_fc2, W_out, b_out):
    custT = encoded_customers.T
    prodT = encoded_products.T
    Wbig_c, bbig_c = _bigw(W_prod, b_prod)
    Wbig_p, bbig_p = _bigw(W_cust, b_cust)
    zc, zp = _scan(custT, prodT, Wbig_c, bbig_c, Wbig_p, bbig_p)

    rp = X[:, 0].astype(jnp.int32)
    rc = X[:, 1].astype(jnp.int32)
    ip = (rp & (CMOD - 1)).reshape(NW, NCHUNK, CHUNK)
    ic = (rc & (CMOD - 1)).reshape(NW, NCHUNK, CHUNK)
    pp = (rp >> 17).reshape(BATCH, 1)
    pc = (rc >> 17).reshape(BATCH, 1)

    gp, gc = _sc_gather(zc, zp, ip, ic)
    rep = lambda W: jnp.tile(jnp.pad(W, ((0, HPAD - HID), (0, 0))), (PGRP, 1))
    out = _mlp(
        gp, gc, pp, pc,
        rep(W_fc2[:HID]), rep(W_fc2[HID:]), b_fc2.reshape(1, HID),
        W_out, b_out.reshape(1, 1),
    )
    return out
